# baseline (speedup 1.0000x reference)
import functools

import jax
import jax.numpy as jnp
from jax import lax
from jax.experimental import pallas as pl
from jax.experimental.pallas import tpu as pltpu
from jax.experimental.pallas import tpu_sc as plsc

_B = 16
_D = 1024
_H = 65536
_K = 50
_DECAY = 0.95
_BK = 2048
_M = 1024
_L = 16
_IMIN = -2147483648


def _mm_kernel(x_ref, w_ref, out_ref):
    x = x_ref[...]
    mn = jnp.min(x)
    mx = jnp.max(x)
    xn = (x - mn) / (mx - mn)
    w = w_ref[...]
    out_ref[...] = jax.lax.dot_general(
        xn, w, (((1,), (1,)), ((), ())), preferred_element_type=jnp.float32
    )


def _lane():
    return lax.broadcasted_iota(jnp.int32, (_L,), 0)


def _radix_select(loader, nchunks, hist_ref, pool0, k):
    lane = _lane()
    zero16 = jnp.zeros((_L,), jnp.int32)
    one16 = jnp.ones((_L,), jnp.int32)
    P = jnp.uint32(0)
    need = jnp.int32(k)
    pool = jnp.int32(pool0)

    for rnd in range(4):
        shift = jnp.uint32(24 - 8 * rnd)

        def zh(j, c):
            plsc.store_scatter(hist_ref, [j * _L + lane], zero16)
            return c

        lax.fori_loop(0, 256, zh, jnp.int32(0))

        def acc(j, c, _rnd=rnd, _shift=shift, _P=P):
            u = loader(j)
            if _rnd == 0:
                ok = lane >= 0
            else:
                ok = (u >> jnp.uint32(_shift + 8)) == _P
            b = ((u >> _shift) & jnp.uint32(255)).astype(jnp.int32)
            plsc.addupdate_scatter(hist_ref, [lane * 256 + b], one16, mask=ok)
            return c

        lax.fori_loop(0, nchunks, acc, jnp.int32(0))

        thresh = pool - need

        def hsc(h, carry):
            run, nsat, s1, m1 = carry
            tot = zero16
            for l in range(_L):
                tot = tot + plsc.load_gather(hist_ref, [l * 256 + h * _L + lane])
            incl = plsc.cumsum(tot)
            pe = run + incl - tot
            cond = pe <= thresh
            nsat = nsat + jnp.sum(cond.astype(jnp.int32))
            s1 = s1 + jnp.sum(jnp.where(cond, tot, 0))
            m1 = jnp.maximum(m1, jnp.max(jnp.where(cond, pe, 0)))
            run = run + jnp.sum(tot)
            return run, nsat, s1, m1

        z = jnp.int32(0)
        _, nsat, s1, m1 = lax.fori_loop(0, 16, hsc, (z, z, z, z))
        bstar = nsat - 1
        need = need - (pool - s1)
        pool = s1 - m1
        P = (P << jnp.uint32(8)) | bstar.astype(jnp.uint32)

    return P, need


def _sc_body(enc_hbm, out_hbm, ebuf, hist, candv, candi, cu, shv, shi,
             acv, aci, hidx, fidx, fsrc, sem):
    c = lax.axis_index("c")
    s = lax.axis_index("s")
    lane = _lane()
    zero16f = jnp.zeros((_L,), jnp.float32)
    nch = _H // _L

    @pl.when(c == 0)
    def _phase1():
        base = s * _H
        pltpu.sync_copy(enc_hbm.at[pl.ds(base, _H)], ebuf)

        def load_enc(j):
            v = plsc.load_gather(ebuf, [j * _L + lane])
            return lax.bitcast_convert_type(v, jnp.uint32)

        T, need = _radix_select(load_enc, nch, hist, _H, _M)

        def cp(j, carry):
            run, off = carry
            v = plsc.load_gather(ebuf, [j * _L + lane])
            u = lax.bitcast_convert_type(v, jnp.uint32)
            gt = u > T
            eq = u == T
            hit = jnp.any(gt | eq)

            def do(carry2):
                run2, off2 = carry2
                eqc = plsc.cumsum(eq.astype(jnp.int32))
                rank = run2 + eqc - 1
                take = gt | (eq & (rank < need))
                tc = plsc.cumsum(take.astype(jnp.int32))
                pos = jnp.maximum(off2 + tc - 1, 0)
                plsc.store_scatter(candv, [pos], v, mask=take)
                plsc.store_scatter(candi, [pos], j * _L + lane, mask=take)
                return (run2 + jnp.sum(eq.astype(jnp.int32)),
                        off2 + jnp.sum(take.astype(jnp.int32)))

            return lax.cond(hit, do, lambda carry2: carry2, carry)

        lax.fori_loop(0, nch, cp, (jnp.int32(0), jnp.int32(0)))

        def zo(j, cc):
            plsc.store_scatter(ebuf, [j * _L + lane], zero16f)
            return cc

        lax.fori_loop(0, nch, zo, jnp.int32(0))
        pltpu.sync_copy(ebuf, out_hbm.at[pl.ds(base, _H)])

        pltpu.sync_copy(candv, shv.at[pl.ds(s * _M, _M)])
        pltpu.sync_copy(candi, shi.at[pl.ds(s * _M, _M)])

    plsc.subcore_barrier()

    @pl.when((c == 0) & (s == 0))
    def _phase2():
        pltpu.sync_copy(shv, acv)
        pltpu.sync_copy(shi, aci)

        lane0 = _lane()
        for j in range(4):
            plsc.store_scatter(fsrc, [j * _L + lane0], jnp.ones((_L,), jnp.float32))

        def zh(j, cc):
            plsc.store_scatter(hidx, [j * _L + lane], jnp.zeros((_L,), jnp.int32))
            return cc

        lax.fori_loop(0, (_B * _K + _L - 1) // _L, zh, jnp.int32(0))

        cch = _M // _L

        def row(i, hsize):
            def mk(j, cc):
                k = i * _M + j * _L + lane
                cvv = plsc.load_gather(acv, [k])
                cii = plsc.load_gather(aci, [k])
                ih = plsc.load_gather(ebuf, [cii])
                refr = cvv * (1.0 - ih)
                bits = lax.bitcast_convert_type(refr, jnp.int32)
                m = jnp.where(bits < 0, _IMIN - bits, bits) ^ _IMIN
                plsc.store_scatter(cu, [j * _L + lane], m)
                return cc

            lax.fori_loop(0, cch, mk, jnp.int32(0))

            def load_cu(j):
                v = plsc.load_gather(cu, [j * _L + lane])
                return lax.bitcast_convert_type(v, jnp.uint32)

            T2, need2 = _radix_select(load_cu, cch, hist, _M, _K)

            def fin(j, carry):
                run, off = carry
                u = load_cu(j)
                gt = u > T2
                eq = u == T2
                eqc = plsc.cumsum(eq.astype(jnp.int32))
                rank = run + eqc - 1
                take = gt | (eq & (rank < need2))
                tc = plsc.cumsum(take.astype(jnp.int32))
                pos = jnp.maximum(off + tc - 1, 0)
                cii = plsc.load_gather(aci, [i * _M + j * _L + lane])
                plsc.store_scatter(fidx, [pos], cii + i * _H, mask=take)
                return (run + jnp.sum(eq.astype(jnp.int32)),
                        off + jnp.sum(take.astype(jnp.int32)))

            lax.fori_loop(0, cch, fin, (jnp.int32(0), jnp.int32(0)))

            f0 = plsc.load_gather(fidx, [jnp.zeros((_L,), jnp.int32)])
            padp = 48 + lane
            plsc.store_scatter(fidx, [padp], f0,
                               mask=(padp >= _K) & (padp < 64))

            def dec(j, cc):
                p = j * _L + lane
                msk = p < hsize
                hu = plsc.load_gather(hidx, [p], mask=msk)
                val = plsc.load_gather(ebuf, [hu], mask=msk)
                plsc.store_scatter(ebuf, [hu], val * _DECAY, mask=msk)
                return cc

            lax.fori_loop(0, (_B * _K) // _L, dec, jnp.int32(0))

            def upd(j, hs):
                p = j * _L + lane
                msk = p < _K
                un = plsc.load_gather(fidx, [p], mask=msk) - i * _H
                un = jnp.maximum(un, 0)
                ih = plsc.load_gather(ebuf, [un], mask=msk)
                newm = msk & (ih == 0.0)
                plsc.store_scatter(ebuf, [un], ih + 1.0, mask=msk)
                nc = plsc.cumsum(newm.astype(jnp.int32))
                slot = jnp.maximum(hs + nc - 1, 0)
                plsc.store_scatter(hidx, [slot], un, mask=newm)
                return hs + jnp.sum(newm.astype(jnp.int32))

            hsize = lax.fori_loop(0, (_K + _L - 1) // _L, upd, hsize)

            pltpu.async_copy(fsrc, out_hbm.at[fidx], sem).wait()
            return hsize

        lax.fori_loop(0, _B, row, jnp.int32(0))


def _sc_scan(encoding):
    mesh = plsc.VectorSubcoreMesh(core_axis_name="c", subcore_axis_name="s")
    fn = pl.kernel(
        _sc_body,
        out_type=jax.ShapeDtypeStruct((_B * _H,), jnp.float32),
        mesh=mesh,
        compiler_params=pltpu.CompilerParams(needs_layout_passes=False),
        scratch_types=[
            pltpu.VMEM((_H,), jnp.float32),
            pltpu.VMEM((4096,), jnp.int32),
            pltpu.VMEM((_M,), jnp.float32),
            pltpu.VMEM((_M,), jnp.int32),
            pltpu.VMEM((_M,), jnp.int32),
            pltpu.VMEM_SHARED((_B * _M,), jnp.float32),
            pltpu.VMEM_SHARED((_B * _M,), jnp.int32),
            pltpu.VMEM((_B * _M,), jnp.float32),
            pltpu.VMEM((_B * _M,), jnp.int32),
            pltpu.VMEM((_B * _K,), jnp.int32),
            pltpu.VMEM((64,), jnp.int32),
            pltpu.VMEM((64,), jnp.float32),
            pltpu.SemaphoreType.DMA,
        ],
    )
    return fn(encoding.reshape(_B * _H))


def kernel(inputs, W):
    x = inputs.reshape(_B, -1)

    encoding = pl.pallas_call(
        _mm_kernel,
        grid=(_H // _BK,),
        in_specs=[
            pl.BlockSpec((_B, _D), lambda i: (0, 0)),
            pl.BlockSpec((_BK, _D), lambda i: (i, 0)),
        ],
        out_specs=pl.BlockSpec((_B, _BK), lambda i: (0, i)),
        out_shape=jax.ShapeDtypeStruct((_B, _H), jnp.float32),
        compiler_params=pltpu.CompilerParams(
            dimension_semantics=("arbitrary",),
        ),
    )(x, W)

    return _sc_scan(encoding).reshape(_B, _H)


# speedup vs baseline: 1.1766x; 1.1766x over previous
import functools

import jax
import jax.numpy as jnp
from jax import lax
from jax.experimental import pallas as pl
from jax.experimental.pallas import tpu as pltpu
from jax.experimental.pallas import tpu_sc as plsc

_B = 16
_D = 1024
_H = 65536
_K = 50
_DECAY = 0.95
_BK = 2048
_M = 1024
_L = 16
_IMIN = -2147483648


def _mm_kernel(x_ref, w_ref, out_ref):
    x = x_ref[...]
    mn = jnp.min(x)
    mx = jnp.max(x)
    xn = (x - mn) / (mx - mn)
    w = w_ref[...]
    out_ref[...] = jax.lax.dot_general(
        xn, w, (((1,), (1,)), ((), ())), preferred_element_type=jnp.float32
    )


def _lane():
    return lax.broadcasted_iota(jnp.int32, (_L,), 0)


def _radix_select(loader, nchunks, hist_ref, pool0, k, unroll=4):
    lane = _lane()
    zero16 = jnp.zeros((_L,), jnp.int32)
    one16 = jnp.ones((_L,), jnp.int32)
    P = jnp.uint32(0)
    need = jnp.int32(k)
    pool = jnp.int32(pool0)

    for rnd in range(4):
        shift = jnp.uint32(24 - 8 * rnd)

        for h in range(16):
            plsc.store_scatter(hist_ref, [h * _L + lane], zero16)

        def acc(jj, c, _rnd=rnd, _shift=shift, _P=P):
            for t in range(unroll):
                u = loader(jj * unroll + t)
                b = ((u >> _shift) & jnp.uint32(255)).astype(jnp.int32)
                if _rnd == 0:
                    plsc.addupdate_scatter(hist_ref, [b], one16)
                else:
                    ok = (u >> jnp.uint32(_shift + 8)) == _P
                    plsc.addupdate_scatter(hist_ref, [b], one16, mask=ok)
            return c

        lax.fori_loop(0, nchunks // unroll, acc, jnp.int32(0))

        thresh = pool - need
        tots = [plsc.load_gather(hist_ref, [h * _L + lane]) for h in range(16)]
        bsums = [jnp.sum(t) for t in tots]
        run = jnp.int32(0)
        nsat_v = zero16
        s1_v = zero16
        m1_v = zero16
        for h in range(16):
            tot = tots[h]
            incl = plsc.cumsum(tot)
            pe = run + incl - tot
            cond = pe <= thresh
            nsat_v = nsat_v + cond.astype(jnp.int32)
            s1_v = s1_v + jnp.where(cond, tot, 0)
            m1_v = jnp.maximum(m1_v, jnp.where(cond, pe, 0))
            run = run + bsums[h]
        nsat = jnp.sum(nsat_v)
        s1 = jnp.sum(s1_v)
        m1 = jnp.max(m1_v)
        bstar = nsat - 1
        need = need - (pool - s1)
        pool = s1 - m1
        P = (P << jnp.uint32(8)) | bstar.astype(jnp.uint32)

    return P, need


def _sc_body(enc_hbm, out_hbm, ebuf, hist, candv, candi, cu, shv, shi,
             acv, aci, hidx, gfid, fsrc, sem):
    c = lax.axis_index("c")
    s = lax.axis_index("s")
    lane = _lane()
    zero16f = jnp.zeros((_L,), jnp.float32)
    nch = _H // _L

    @pl.when(c == 0)
    def _phase1():
        base = s * _H
        pltpu.sync_copy(enc_hbm.at[pl.ds(base, _H)], ebuf)

        def load_enc(j):
            v = plsc.load_gather(ebuf, [j * _L + lane])
            return lax.bitcast_convert_type(v, jnp.uint32)

        T, need = _radix_select(load_enc, nch, hist, _H, _M)

        def cp(jj, carry):
            for t in range(2):
                j = jj * 2 + t
                v = plsc.load_gather(ebuf, [j * _L + lane])
                u = lax.bitcast_convert_type(v, jnp.uint32)
                gt = u > T
                eq = u == T
                hit = jnp.any(gt | eq)

                def do(carry2, v=v, gt=gt, eq=eq, j=j):
                    run2, off2 = carry2
                    eqc = plsc.cumsum(eq.astype(jnp.int32))
                    rank = run2 + eqc - 1
                    take = gt | (eq & (rank < need))
                    tc = plsc.cumsum(take.astype(jnp.int32))
                    pos = jnp.maximum(off2 + tc - 1, 0)
                    plsc.store_scatter(candv, [pos], v, mask=take)
                    plsc.store_scatter(candi, [pos], j * _L + lane, mask=take)
                    return (run2 + jnp.sum(eq.astype(jnp.int32)),
                            off2 + jnp.sum(take.astype(jnp.int32)))

                carry = lax.cond(hit, do, lambda carry2: carry2, carry)
                plsc.store_scatter(ebuf, [j * _L + lane], zero16f)
            return carry

        lax.fori_loop(0, nch // 2, cp, (jnp.int32(0), jnp.int32(0)))

        pltpu.sync_copy(ebuf, out_hbm.at[pl.ds(base, _H)])
        pltpu.sync_copy(candv, shv.at[pl.ds(s * _M, _M)])
        pltpu.sync_copy(candi, shi.at[pl.ds(s * _M, _M)])

    plsc.subcore_barrier()

    @pl.when((c == 0) & (s == 0))
    def _phase2():
        pltpu.sync_copy(shv, acv)
        pltpu.sync_copy(shi, aci)

        ones16f = jnp.ones((_L,), jnp.float32)
        for j in range(8):
            plsc.store_scatter(fsrc, [j * _L + lane], ones16f)

        def zh(j, cc):
            plsc.store_scatter(hidx, [j * _L + lane], jnp.zeros((_L,), jnp.int32))
            return cc

        lax.fori_loop(0, (_B * _K + _L - 1) // _L, zh, jnp.int32(0))

        cch = _M // _L

        def gf_write(gpos, val, mask):
            plsc.store_scatter(gfid, [gpos >> 7, gpos & 127], val, mask=mask)

        def gf_read(gpos, mask=None):
            return plsc.load_gather(gfid, [gpos >> 7, gpos & 127], mask=mask)

        def row(i, hsize):
            def mk(jj, cc):
                for t in range(4):
                    j = jj * 4 + t
                    k = i * _M + j * _L + lane
                    cvv = plsc.load_gather(acv, [k])
                    cii = plsc.load_gather(aci, [k])
                    ih = plsc.load_gather(ebuf, [cii])
                    refr = cvv * (1.0 - ih)
                    bits = lax.bitcast_convert_type(refr, jnp.int32)
                    m = jnp.where(bits < 0, _IMIN - bits, bits) ^ _IMIN
                    plsc.store_scatter(cu, [j * _L + lane], m)
                return cc

            lax.fori_loop(0, cch // 4, mk, jnp.int32(0))

            def load_cu(j):
                v = plsc.load_gather(cu, [j * _L + lane])
                return lax.bitcast_convert_type(v, jnp.uint32)

            T2, need2 = _radix_select(load_cu, cch, hist, _M, _K)

            def fin(j, carry):
                run, off = carry
                u = load_cu(j)
                gt = u > T2
                eq = u == T2
                eqc = plsc.cumsum(eq.astype(jnp.int32))
                rank = run + eqc - 1
                take = gt | (eq & (rank < need2))
                tc = plsc.cumsum(take.astype(jnp.int32))
                pos = jnp.maximum(off + tc - 1, 0)
                cii = plsc.load_gather(aci, [i * _M + j * _L + lane])
                gf_write(i * 64 + pos, cii + i * _H, take)
                return (run + jnp.sum(eq.astype(jnp.int32)),
                        off + jnp.sum(take.astype(jnp.int32)))

            lax.fori_loop(0, cch, fin, (jnp.int32(0), jnp.int32(0)))

            f0 = gf_read(jnp.zeros((_L,), jnp.int32) + i * 64)
            padp = 48 + lane
            gf_write(i * 64 + padp, f0, (padp >= _K) & (padp < 64))

            def dec(j, cc):
                p = j * _L + lane
                msk = p < hsize
                hu = plsc.load_gather(hidx, [p], mask=msk)
                val = plsc.load_gather(ebuf, [hu], mask=msk)
                plsc.store_scatter(ebuf, [hu], val * _DECAY, mask=msk)
                return cc

            lax.fori_loop(0, (hsize + _L - 1) >> 4, dec, jnp.int32(0))

            def upd(j, hs):
                p = j * _L + lane
                msk = p < _K
                un = gf_read(i * 64 + p, mask=msk) - i * _H
                un = jnp.maximum(un, 0)
                ih = plsc.load_gather(ebuf, [un], mask=msk)
                newm = msk & (ih == 0.0)
                plsc.store_scatter(ebuf, [un], ih + 1.0, mask=msk)
                nc = plsc.cumsum(newm.astype(jnp.int32))
                slot = jnp.maximum(hs + nc - 1, 0)
                plsc.store_scatter(hidx, [slot], un, mask=newm)
                return hs + jnp.sum(newm.astype(jnp.int32))

            hsize = lax.fori_loop(0, (_K + _L - 1) // _L, upd, hsize)
            return hsize

        lax.fori_loop(0, _B, row, jnp.int32(0))

        handles = [
            pltpu.async_copy(fsrc, out_hbm.at[gfid.at[q]], sem)
            for q in range(8)
        ]
        for h in handles:
            h.wait()


def _sc_scan(encoding):
    mesh = plsc.VectorSubcoreMesh(core_axis_name="c", subcore_axis_name="s")
    fn = pl.kernel(
        _sc_body,
        out_type=jax.ShapeDtypeStruct((_B * _H,), jnp.float32),
        mesh=mesh,
        compiler_params=pltpu.CompilerParams(needs_layout_passes=False),
        scratch_types=[
            pltpu.VMEM((_H,), jnp.float32),
            pltpu.VMEM((256,), jnp.int32),
            pltpu.VMEM((_M,), jnp.float32),
            pltpu.VMEM((_M,), jnp.int32),
            pltpu.VMEM((_M,), jnp.int32),
            pltpu.VMEM_SHARED((_B * _M,), jnp.float32),
            pltpu.VMEM_SHARED((_B * _M,), jnp.int32),
            pltpu.VMEM((_B * _M,), jnp.float32),
            pltpu.VMEM((_B * _M,), jnp.int32),
            pltpu.VMEM((_B * _K,), jnp.int32),
            pltpu.VMEM((8, 128), jnp.int32),
            pltpu.VMEM((128,), jnp.float32),
            pltpu.SemaphoreType.DMA,
        ],
    )
    return fn(encoding.reshape(_B * _H))


def kernel(inputs, W):
    x = inputs.reshape(_B, -1)

    encoding = pl.pallas_call(
        _mm_kernel,
        grid=(_H // _BK,),
        in_specs=[
            pl.BlockSpec((_B, _D), lambda i: (0, 0)),
            pl.BlockSpec((_BK, _D), lambda i: (i, 0)),
        ],
        out_specs=pl.BlockSpec((_B, _BK), lambda i: (0, i)),
        out_shape=jax.ShapeDtypeStruct((_B, _H), jnp.float32),
        compiler_params=pltpu.CompilerParams(
            dimension_semantics=("arbitrary",),
        ),
    )(x, W)

    return _sc_scan(encoding).reshape(_B, _H)


# speedup vs baseline: 2.3624x; 2.0079x over previous
import functools

import jax
import jax.numpy as jnp
from jax import lax
from jax.experimental import pallas as pl
from jax.experimental.pallas import tpu as pltpu
from jax.experimental.pallas import tpu_sc as plsc

_B = 16
_D = 1024
_H = 65536
_K = 50
_DECAY = 0.95
_BK = 2048
_M = 1024
_L = 16
_IMIN = -2147483648


def _mm_kernel(x_ref, w_ref, out_ref):
    x = x_ref[...]
    mn = jnp.min(x)
    mx = jnp.max(x)
    xn = (x - mn) / (mx - mn)
    w = w_ref[...]
    out_ref[...] = jax.lax.dot_general(
        xn, w, (((1,), (1,)), ((), ())), preferred_element_type=jnp.float32
    )


def _lane():
    return lax.broadcasted_iota(jnp.int32, (_L,), 0)


def _radix_select(loader, nchunks, hist_ref, pool0, k, unroll=4):
    lane = _lane()
    zero16 = jnp.zeros((_L,), jnp.int32)
    one16 = jnp.ones((_L,), jnp.int32)
    P = jnp.uint32(0)
    need = jnp.int32(k)
    pool = jnp.int32(pool0)

    for rnd in range(4):
        shift = jnp.uint32(24 - 8 * rnd)

        for h in range(16):
            plsc.store_scatter(hist_ref, [h * _L + lane], zero16)

        def acc(j, _rnd=rnd, _shift=shift, _P=P):
            u = loader(j)
            b = ((u >> _shift) & jnp.uint32(255)).astype(jnp.int32)
            if _rnd == 0:
                plsc.addupdate_scatter(hist_ref, [b], one16)
            else:
                ok = (u >> jnp.uint32(_shift + 8)) == _P
                plsc.addupdate_scatter(hist_ref, [b], one16, mask=ok)

        plsc.parallel_loop(0, nchunks, unroll=unroll)(acc)

        thresh = pool - need
        tots = [plsc.load_gather(hist_ref, [h * _L + lane]) for h in range(16)]
        bsums = [jnp.sum(t) for t in tots]
        run = jnp.int32(0)
        nsat_v = zero16
        s1_v = zero16
        m1_v = zero16
        for h in range(16):
            tot = tots[h]
            incl = plsc.cumsum(tot)
            pe = run + incl - tot
            cond = pe <= thresh
            nsat_v = nsat_v + cond.astype(jnp.int32)
            s1_v = s1_v + jnp.where(cond, tot, 0)
            m1_v = jnp.maximum(m1_v, jnp.where(cond, pe, 0))
            run = run + bsums[h]
        nsat = jnp.sum(nsat_v)
        s1 = jnp.sum(s1_v)
        m1 = jnp.max(m1_v)
        bstar = nsat - 1
        need = need - (pool - s1)
        pool = s1 - m1
        P = (P << jnp.uint32(8)) | bstar.astype(jnp.uint32)

    return P, need


def _sc_body(enc_hbm, out_hbm, ebuf, hist, candv, candi, cu, shv, shi,
             acv, aci, hidx, gfid, fsrc, sem):
    c = lax.axis_index("c")
    s = lax.axis_index("s")
    lane = _lane()
    zero16f = jnp.zeros((_L,), jnp.float32)
    nch = _H // _L

    @pl.when(c == 0)
    def _phase1():
        base = s * _H
        pltpu.sync_copy(enc_hbm.at[pl.ds(base, _H)], ebuf)

        def load_enc(j):
            v = plsc.load_gather(ebuf, [j * _L + lane])
            return lax.bitcast_convert_type(v, jnp.uint32)

        T, need = _radix_select(load_enc, nch, hist, _H, _M)

        def cp(j, carry):
            v = plsc.load_gather(ebuf, [j * _L + lane])
            u = lax.bitcast_convert_type(v, jnp.uint32)
            gt = u > T
            eq = u == T
            hit = jnp.any(gt | eq)

            def do(carry2):
                run2, off2 = carry2
                eqc = plsc.cumsum(eq.astype(jnp.int32))
                rank = run2 + eqc - 1
                take = gt | (eq & (rank < need))
                tc = plsc.cumsum(take.astype(jnp.int32))
                pos = jnp.maximum(off2 + tc - 1, 0)
                plsc.store_scatter(candv, [pos], v, mask=take)
                plsc.store_scatter(candi, [pos], j * _L + lane, mask=take)
                return (run2 + jnp.sum(eq.astype(jnp.int32)),
                        off2 + jnp.sum(take.astype(jnp.int32)))

            carry = lax.cond(hit, do, lambda carry2: carry2, carry)
            plsc.store_scatter(ebuf, [j * _L + lane], zero16f)
            return carry

        plsc.parallel_loop(0, nch, unroll=4,
                           carry=(jnp.int32(0), jnp.int32(0)))(cp)

        pltpu.sync_copy(ebuf, out_hbm.at[pl.ds(base, _H)])
        pltpu.sync_copy(candv, shv.at[pl.ds(s * _M, _M)])
        pltpu.sync_copy(candi, shi.at[pl.ds(s * _M, _M)])

    plsc.subcore_barrier()

    @pl.when((c == 0) & (s == 0))
    def _phase2():
        pltpu.sync_copy(shv, acv)
        pltpu.sync_copy(shi, aci)

        ones16f = jnp.ones((_L,), jnp.float32)
        for j in range(8):
            plsc.store_scatter(fsrc, [j * _L + lane], ones16f)

        def zh(j, cc):
            plsc.store_scatter(hidx, [j * _L + lane], jnp.zeros((_L,), jnp.int32))
            return cc

        lax.fori_loop(0, (_B * _K + _L - 1) // _L, zh, jnp.int32(0))

        cch = _M // _L

        def gf_write(gpos, val, mask):
            plsc.store_scatter(gfid, [gpos >> 7, gpos & 127], val, mask=mask)

        def gf_read(gpos, mask=None):
            return plsc.load_gather(gfid, [gpos >> 7, gpos & 127], mask=mask)

        def row(i, hsize):
            def mk(j):
                k = i * _M + j * _L + lane
                cvv = plsc.load_gather(acv, [k])
                cii = plsc.load_gather(aci, [k])
                ih = plsc.load_gather(ebuf, [cii])
                refr = cvv * (1.0 - ih)
                bits = lax.bitcast_convert_type(refr, jnp.int32)
                m = jnp.where(bits < 0, _IMIN - bits, bits) ^ _IMIN
                plsc.store_scatter(cu, [j * _L + lane], m)

            plsc.parallel_loop(0, cch, unroll=4)(mk)

            def load_cu(j):
                v = plsc.load_gather(cu, [j * _L + lane])
                return lax.bitcast_convert_type(v, jnp.uint32)

            T2, need2 = _radix_select(load_cu, cch, hist, _M, _K)

            def fin(j, carry):
                run, off = carry
                u = load_cu(j)
                gt = u > T2
                eq = u == T2
                eqc = plsc.cumsum(eq.astype(jnp.int32))
                rank = run + eqc - 1
                take = gt | (eq & (rank < need2))
                tc = plsc.cumsum(take.astype(jnp.int32))
                pos = jnp.maximum(off + tc - 1, 0)
                cii = plsc.load_gather(aci, [i * _M + j * _L + lane])
                gf_write(i * 64 + pos, cii + i * _H, take)
                return (run + jnp.sum(eq.astype(jnp.int32)),
                        off + jnp.sum(take.astype(jnp.int32)))

            plsc.parallel_loop(0, cch, unroll=4,
                               carry=(jnp.int32(0), jnp.int32(0)))(fin)

            f0 = gf_read(jnp.zeros((_L,), jnp.int32) + i * 64)
            padp = 48 + lane
            gf_write(i * 64 + padp, f0, (padp >= _K) & (padp < 64))

            def dec(j, cc):
                p = j * _L + lane
                msk = p < hsize
                hu = plsc.load_gather(hidx, [p], mask=msk)
                val = plsc.load_gather(ebuf, [hu], mask=msk)
                plsc.store_scatter(ebuf, [hu], val * _DECAY, mask=msk)
                return cc

            lax.fori_loop(0, (hsize + _L - 1) >> 4, dec, jnp.int32(0))

            def upd(j, hs):
                p = j * _L + lane
                msk = p < _K
                un = gf_read(i * 64 + p, mask=msk) - i * _H
                un = jnp.maximum(un, 0)
                ih = plsc.load_gather(ebuf, [un], mask=msk)
                newm = msk & (ih == 0.0)
                plsc.store_scatter(ebuf, [un], ih + 1.0, mask=msk)
                nc = plsc.cumsum(newm.astype(jnp.int32))
                slot = jnp.maximum(hs + nc - 1, 0)
                plsc.store_scatter(hidx, [slot], un, mask=newm)
                return hs + jnp.sum(newm.astype(jnp.int32))

            hsize = lax.fori_loop(0, (_K + _L - 1) // _L, upd, hsize)
            return hsize

        lax.fori_loop(0, _B, row, jnp.int32(0))

        handles = [
            pltpu.async_copy(fsrc, out_hbm.at[gfid.at[q]], sem)
            for q in range(8)
        ]
        for h in handles:
            h.wait()


def _sc_scan(encoding):
    mesh = plsc.VectorSubcoreMesh(core_axis_name="c", subcore_axis_name="s")
    fn = pl.kernel(
        _sc_body,
        out_type=jax.ShapeDtypeStruct((_B * _H,), jnp.float32),
        mesh=mesh,
        compiler_params=pltpu.CompilerParams(needs_layout_passes=False),
        scratch_types=[
            pltpu.VMEM((_H,), jnp.float32),
            pltpu.VMEM((256,), jnp.int32),
            pltpu.VMEM((_M,), jnp.float32),
            pltpu.VMEM((_M,), jnp.int32),
            pltpu.VMEM((_M,), jnp.int32),
            pltpu.VMEM_SHARED((_B * _M,), jnp.float32),
            pltpu.VMEM_SHARED((_B * _M,), jnp.int32),
            pltpu.VMEM((_B * _M,), jnp.float32),
            pltpu.VMEM((_B * _M,), jnp.int32),
            pltpu.VMEM((_B * _K,), jnp.int32),
            pltpu.VMEM((8, 128), jnp.int32),
            pltpu.VMEM((128,), jnp.float32),
            pltpu.SemaphoreType.DMA,
        ],
    )
    return fn(encoding.reshape(_B * _H))


def kernel(inputs, W):
    x = inputs.reshape(_B, -1)

    encoding = pl.pallas_call(
        _mm_kernel,
        grid=(_H // _BK,),
        in_specs=[
            pl.BlockSpec((_B, _D), lambda i: (0, 0)),
            pl.BlockSpec((_BK, _D), lambda i: (i, 0)),
        ],
        out_specs=pl.BlockSpec((_B, _BK), lambda i: (0, i)),
        out_shape=jax.ShapeDtypeStruct((_B, _H), jnp.float32),
        compiler_params=pltpu.CompilerParams(
            dimension_semantics=("arbitrary",),
        ),
    )(x, W)

    return _sc_scan(encoding).reshape(_B, _H)
